# baseline (speedup 1.0000x reference)
import functools

import jax
import jax.numpy as jnp
from jax import lax
from jax.experimental import pallas as pl
from jax.experimental.pallas import tpu as pltpu
from jax.experimental.pallas import tpu_sc as plsc

_N = 1048576
_F = 524288
_NC = 2
_NS = 16
_NW = _NC * _NS
_L = 16
_C = (_N - _F) // _NW
_HP = 4
_P = _C // _HP
_PV = _P // _L
_U = 8

_mesh = plsc.VectorSubcoreMesh(core_axis_name="c", subcore_axis_name="s")


@functools.partial(
    pl.kernel,
    mesh=_mesh,
    out_type=jax.ShapeDtypeStruct((_NW, _L), jnp.float32),
    scratch_types=[
        pltpu.VMEM((_C + _L,), jnp.float32),
        pltpu.VMEM((_C + _L,), jnp.float32),
        pltpu.VMEM((_L,), jnp.float32),
        pltpu.SemaphoreType.DMA,
        pltpu.SemaphoreType.DMA,
    ],
)
def _sc_partial_sums(yt_hbm, yp_hbm, out_hbm, bt, bp, acc_v, sem0, sem1):
    cid = lax.axis_index("c")
    sid = lax.axis_index("s")
    wid = sid * _NC + cid
    start = pl.multiple_of(_F + wid * _C, _C)
    sems = (sem0, sem1)

    copies = [
        pltpu.async_copy(yt_hbm.at[pl.ds(start - _L, _L)], bt.at[pl.ds(0, _L)], sem0),
        pltpu.async_copy(yp_hbm.at[pl.ds(start - _L, _L)], bp.at[pl.ds(0, _L)], sem0),
    ]

    def start_piece(p):
        off = pl.multiple_of(start + p * _P, _P)
        s = sems[p % 2]
        return [
            pltpu.async_copy(yt_hbm.at[pl.ds(off, _P)], bt.at[pl.ds(p * _P + _L, _P)], s),
            pltpu.async_copy(yp_hbm.at[pl.ds(off, _P)], bp.at[pl.ds(p * _P + _L, _P)], s),
        ]

    copies += start_piece(0)
    for c in copies:
        c.wait()

    total = jnp.zeros((_L,), jnp.float32)
    carry_p = bp[pl.ds(0, _L)]
    rot_idx = (lax.iota(jnp.int32, _L) + (_L - 1)) % _L
    for p in range(_HP):
        nxt = start_piece(p + 1) if p + 1 < _HP else []

        @plsc.parallel_loop(p * _PV, (p + 1) * _PV, step=_U, carry=(total, carry_p))
        def acc(i, carry_in):
            acc_in, pc = carry_in
            lane = lax.iota(jnp.int32, _L)
            terms = []
            for u in range(_U):
                base = (i + u) * _L
                cur_t = bt[pl.ds(base + _L, _L)]
                cur_p = bp[pl.ds(base + _L, _L)]
                prev_t = bt[pl.ds(base + (_L - 1), _L)]
                merged = jnp.where(lane == _L - 1, pc, cur_p)
                prev_p = merged.at[rot_idx].get(
                    mode=lax.GatherScatterMode.PROMISE_IN_BOUNDS)
                pc = cur_p
                mm = jnp.logical_xor(cur_t > prev_t, cur_p > prev_p)
                d = cur_p - cur_t
                se = d * d
                terms.append(jnp.where(mm, se * 10000.0, se))
            while len(terms) > 1:
                terms = [a + b for a, b in zip(terms[::2], terms[1::2])]
            return acc_in + terms[0], pc

        total, carry_p = acc
        for c in nxt:
            c.wait()

    acc_v[...] = total
    pltpu.sync_copy(acc_v, out_hbm.at[wid])


_ALLROWS = _N // 128
_ROWS = _F // 128
_BR = 1024
_STEPS = _ROWS // _BR


def _tc_body(yt_ref, yp_ref, ht_ref, hp_ref, out_ref):
    i = pl.program_id(0)
    t = yt_ref[...]
    p = yp_ref[...]
    halo_t = jnp.where(i == 0, t[0:1, 0:1], ht_ref[7:8, 127:128])
    halo_p = jnp.where(i == 0, p[0:1, 0:1], hp_ref[7:8, 127:128])
    col_t = jnp.concatenate([halo_t, t[:-1, 127:128]], axis=0)
    col_p = jnp.concatenate([halo_p, p[:-1, 127:128]], axis=0)
    prev_t = jnp.concatenate([col_t, t[:, :127]], axis=1)
    prev_p = jnp.concatenate([col_p, p[:, :127]], axis=1)
    mm = jnp.logical_xor(t > prev_t, p > prev_p)
    d = p - t
    se = d * d
    w = jnp.where(mm, se * 10000.0, se)
    partial = jnp.sum(w)

    @pl.when(i == 0)
    def _init():
        out_ref[0, 0] = 0.0

    out_ref[0, 0] += partial


_tc_partial = pl.pallas_call(
    _tc_body,
    grid=(_STEPS,),
    in_specs=[
        pl.BlockSpec((_BR, 128), lambda i: (i, 0)),
        pl.BlockSpec((_BR, 128), lambda i: (i, 0)),
        pl.BlockSpec((8, 128), lambda i: (jnp.maximum(i * (_BR // 8) - 1, 0), 0)),
        pl.BlockSpec((8, 128), lambda i: (jnp.maximum(i * (_BR // 8) - 1, 0), 0)),
    ],
    out_specs=pl.BlockSpec(memory_space=pltpu.SMEM),
    out_shape=jax.ShapeDtypeStruct((1, 1), jnp.float32),
)


def kernel(y_true, y_pred):
    yt2d = y_true.reshape(_ALLROWS, 128)
    yp2d = y_pred.reshape(_ALLROWS, 128)
    tc_sum = _tc_partial(yt2d, yp2d, yt2d, yp2d)
    sc_partials = _sc_partial_sums(y_true, y_pred)
    return (tc_sum[0, 0] + jnp.sum(sc_partials)) / jnp.float32(_N)


# speedup vs baseline: 1.0456x; 1.0456x over previous
import functools

import jax
import jax.numpy as jnp
from jax import lax
from jax.experimental import pallas as pl
from jax.experimental.pallas import tpu as pltpu
from jax.experimental.pallas import tpu_sc as plsc

_N = 1048576
_F = 524288
_NC = 2
_NS = 16
_NW = _NC * _NS
_L = 16
_C = (_N - _F) // _NW
_HP = 2
_P = _C // _HP
_PV = _P // _L
_U = 8

_mesh = plsc.VectorSubcoreMesh(core_axis_name="c", subcore_axis_name="s")


@functools.partial(
    pl.kernel,
    mesh=_mesh,
    out_type=jax.ShapeDtypeStruct((_NW, _L), jnp.float32),
    scratch_types=[
        pltpu.VMEM((_C + _L,), jnp.float32),
        pltpu.VMEM((_C + _L,), jnp.float32),
        pltpu.VMEM((_L,), jnp.float32),
        pltpu.SemaphoreType.DMA,
        pltpu.SemaphoreType.DMA,
    ],
)
def _sc_partial_sums(yt_hbm, yp_hbm, out_hbm, bt, bp, acc_v, sem0, sem1):
    cid = lax.axis_index("c")
    sid = lax.axis_index("s")
    wid = sid * _NC + cid
    start = pl.multiple_of(_F + wid * _C, _C)
    sems = (sem0, sem1)

    copies = [
        pltpu.async_copy(yt_hbm.at[pl.ds(start - _L, _L)], bt.at[pl.ds(0, _L)], sem0),
        pltpu.async_copy(yp_hbm.at[pl.ds(start - _L, _L)], bp.at[pl.ds(0, _L)], sem0),
    ]

    def start_piece(p):
        off = pl.multiple_of(start + p * _P, _P)
        s = sems[p % 2]
        return [
            pltpu.async_copy(yt_hbm.at[pl.ds(off, _P)], bt.at[pl.ds(p * _P + _L, _P)], s),
            pltpu.async_copy(yp_hbm.at[pl.ds(off, _P)], bp.at[pl.ds(p * _P + _L, _P)], s),
        ]

    copies += start_piece(0)
    for c in copies:
        c.wait()

    total = jnp.zeros((_L,), jnp.float32)
    carry_p = bp[pl.ds(0, _L)]
    rot_idx = (lax.iota(jnp.int32, _L) + (_L - 1)) % _L
    for p in range(_HP):
        nxt = start_piece(p + 1) if p + 1 < _HP else []

        @plsc.parallel_loop(p * _PV, (p + 1) * _PV, step=_U, carry=(total, carry_p))
        def acc(i, carry_in):
            acc_in, pc = carry_in
            lane = lax.iota(jnp.int32, _L)
            terms = []
            for u in range(_U):
                base = (i + u) * _L
                cur_t = bt[pl.ds(base + _L, _L)]
                cur_p = bp[pl.ds(base + _L, _L)]
                prev_t = bt[pl.ds(base + (_L - 1), _L)]
                merged = jnp.where(lane == _L - 1, pc, cur_p)
                prev_p = merged.at[rot_idx].get(
                    mode=lax.GatherScatterMode.PROMISE_IN_BOUNDS)
                pc = cur_p
                mm = jnp.logical_xor(cur_t > prev_t, cur_p > prev_p)
                d = cur_p - cur_t
                se = d * d
                terms.append(jnp.where(mm, se * 10000.0, se))
            while len(terms) > 1:
                terms = [a + b for a, b in zip(terms[::2], terms[1::2])]
            return acc_in + terms[0], pc

        total, carry_p = acc
        for c in nxt:
            c.wait()

    acc_v[...] = total
    pltpu.sync_copy(acc_v, out_hbm.at[wid])


_ALLROWS = _N // 128
_ROWS = _F // 128
_BR = 1024
_STEPS = _ROWS // _BR


def _tc_body(yt_ref, yp_ref, ht_ref, hp_ref, out_ref):
    i = pl.program_id(0)
    t = yt_ref[...]
    p = yp_ref[...]
    halo_t = jnp.where(i == 0, t[0:1, 0:1], ht_ref[7:8, 127:128])
    halo_p = jnp.where(i == 0, p[0:1, 0:1], hp_ref[7:8, 127:128])
    col_t = jnp.concatenate([halo_t, t[:-1, 127:128]], axis=0)
    col_p = jnp.concatenate([halo_p, p[:-1, 127:128]], axis=0)
    prev_t = jnp.concatenate([col_t, t[:, :127]], axis=1)
    prev_p = jnp.concatenate([col_p, p[:, :127]], axis=1)
    mm = jnp.logical_xor(t > prev_t, p > prev_p)
    d = p - t
    se = d * d
    w = jnp.where(mm, se * 10000.0, se)
    partial = jnp.sum(w)

    @pl.when(i == 0)
    def _init():
        out_ref[0, 0] = 0.0

    out_ref[0, 0] += partial


_tc_partial = pl.pallas_call(
    _tc_body,
    grid=(_STEPS,),
    in_specs=[
        pl.BlockSpec((_BR, 128), lambda i: (i, 0)),
        pl.BlockSpec((_BR, 128), lambda i: (i, 0)),
        pl.BlockSpec((8, 128), lambda i: (jnp.maximum(i * (_BR // 8) - 1, 0), 0)),
        pl.BlockSpec((8, 128), lambda i: (jnp.maximum(i * (_BR // 8) - 1, 0), 0)),
    ],
    out_specs=pl.BlockSpec(memory_space=pltpu.SMEM),
    out_shape=jax.ShapeDtypeStruct((1, 1), jnp.float32),
)


def kernel(y_true, y_pred):
    yt2d = y_true.reshape(_ALLROWS, 128)
    yp2d = y_pred.reshape(_ALLROWS, 128)
    tc_sum = _tc_partial(yt2d, yp2d, yt2d, yp2d)
    sc_partials = _sc_partial_sums(y_true, y_pred)
    return (tc_sum[0, 0] + jnp.sum(sc_partials)) / jnp.float32(_N)
